# baseline (speedup 1.0000x reference)
import functools

import jax
import jax.numpy as jnp
from jax import lax
from jax.experimental import pallas as pl
from jax.experimental.pallas import tpu as pltpu
from jax.experimental.pallas import tpu_sc as plsc

N = 10000
E = 320000
D = 128

NC = 2
NS = 16
NW = NC * NS
CHUNK = 80
NZCH = N // CHUNK

NCH_A = 63
NCH_B = 62
E_A = NW * NCH_A * CHUNK
E_B = E - E_A

BR = 2560


def _proj_body(x_ref, w_ref, b_ref, o_ref):
    o_ref[...] = jnp.dot(x_ref[...], w_ref[...],
                         preferred_element_type=jnp.float32) + b_ref[...]


def _node_proj(x, w_top, b_en):
    return pl.pallas_call(
        _proj_body,
        out_shape=jax.ShapeDtypeStruct((N, D), jnp.float32),
    )(x, w_top, b_en)


NB = 4


def _ring(nchunk, fire_in, wait_in, fire_out, wait_out):
    for b in range(NB - 1):
        fire_in(b, b)
    full = nchunk // NB
    rem = nchunk % NB

    def step(j, u, i):
        wait_in(u)
        fire_out(j, u)
        pb = (u - 1) % NB
        if u == 0:
            if i is None:
                wait_out(pb)
            else:
                @pl.when(i > 0)
                def _():
                    wait_out(pb)
        else:
            wait_out(pb)
        if i is None:
            if j + NB - 1 < nchunk:
                fire_in(j + NB - 1, pb)
        elif u < rem:
            fire_in(j + NB - 1, pb)
        else:
            @pl.when(j + NB - 1 < nchunk)
            def _():
                fire_in(j + NB - 1, pb)

    def body(i, carry):
        for u in range(NB):
            step(i * NB + u, u, i)
        return carry

    lax.fori_loop(0, full, body, 0)
    for u in range(rem):
        step(full * NB + u, u, None)
    wait_out((nchunk - 1) % NB)


@functools.cache
def _make_sc_gather(nchunk):
    mesh = plsc.VectorSubcoreMesh(core_axis_name="c", subcore_axis_name="s")
    epw = nchunk * CHUNK

    @functools.partial(
        pl.kernel,
        mesh=mesh,
        out_type=jax.ShapeDtypeStruct((NW * epw, D), jnp.float32),
        scratch_types=[
            pltpu.VMEM((nchunk, CHUNK), jnp.int32),
            pltpu.VMEM((NB, CHUNK, D), jnp.float32),
            pltpu.VMEM_SHARED((N, D), jnp.float32),
        ] + [pltpu.SemaphoreType.DMA] * (2 * NB),
    )
    def _sc_gather(p_hbm, row3_hbm, out_hbm, idx2d, rows_v, tab_sh, *sems):
        s = lax.axis_index("s")
        wid = s * NC + lax.axis_index("c")
        base = wid * epw
        gsem = sems[:NB]
        ssem = sems[NB:]

        nz = jnp.where(s < NZCH % NS, NZCH // NS + 1, NZCH // NS)

        def st(k, carry):
            r0 = (s + k * NS) * CHUNK
            pltpu.async_copy(p_hbm.at[pl.ds(r0, CHUNK)],
                             tab_sh.at[pl.ds(r0, CHUNK)], gsem[0])
            return carry

        lax.fori_loop(0, nz, st, 0)

        def stw(k, carry):
            pltpu.make_async_copy(p_hbm.at[pl.ds(0, CHUNK)],
                                  tab_sh.at[pl.ds(0, CHUNK)], gsem[0]).wait()
            return carry

        lax.fori_loop(0, nz, stw, 0)
        plsc.subcore_barrier()

        def fire_g(j, b):
            pltpu.async_copy(tab_sh.at[idx2d.at[j]], rows_v.at[b], gsem[b])

        def wait_g(b):
            pltpu.make_async_copy(tab_sh.at[idx2d.at[0]], rows_v.at[b],
                                  gsem[b]).wait()

        def fire_s(j, b):
            pltpu.async_copy(rows_v.at[b],
                             out_hbm.at[pl.ds(base + j * CHUNK, CHUNK)], ssem[b])

        def wait_s(b):
            pltpu.make_async_copy(rows_v.at[b], out_hbm.at[pl.ds(base, CHUNK)],
                                  ssem[b]).wait()

        pltpu.sync_copy(row3_hbm.at[wid], idx2d)
        _ring(nchunk, fire_g, wait_g, fire_s, wait_s)

    return _sc_gather


def _edge_body(g_ref, a_ref, w_ref, g1_ref, b1_ref, o_ref):
    t = g_ref[...] + jnp.dot(a_ref[...], w_ref[...],
                             preferred_element_type=jnp.float32)
    jm = jnp.full((D, D), 1.0 / D, dtype=jnp.float32)
    mu = jnp.dot(t, jm, preferred_element_type=jnp.float32)
    m2 = jnp.dot(t * t, jm, preferred_element_type=jnp.float32)
    var = m2 - mu * mu
    t = (t - mu) / jnp.sqrt(var + 1e-5) * g1_ref[...] + b1_ref[...]
    o_ref[...] = jnp.maximum(t, 0.0)


def _edge_mlp(g, a, w_bot, g1, b1, steps, off):
    return pl.pallas_call(
        _edge_body,
        grid=(steps,),
        in_specs=[
            pl.BlockSpec((BR, D), lambda i: (i, 0)),
            pl.BlockSpec((BR, D), lambda i: (i + off, 0)),
            pl.BlockSpec((D, D), lambda i: (0, 0)),
            pl.BlockSpec((1, D), lambda i: (0, 0)),
            pl.BlockSpec((1, D), lambda i: (0, 0)),
        ],
        out_specs=pl.BlockSpec((BR, D), lambda i: (i, 0)),
        out_shape=jax.ShapeDtypeStruct((steps * BR, D), jnp.float32),
    )(g, a, w_bot, g1, b1)


@functools.cache
def _make_sc_scatter(nchunk):
    mesh = plsc.VectorSubcoreMesh(core_axis_name="c", subcore_axis_name="s")
    epw = nchunk * CHUNK

    @functools.partial(
        pl.kernel,
        mesh=mesh,
        out_type=jax.ShapeDtypeStruct((NC, N, D), jnp.float32),
        scratch_types=[
            pltpu.VMEM((nchunk, CHUNK), jnp.int32),
            pltpu.VMEM((NB, CHUNK, D), jnp.float32),
            pltpu.VMEM_SHARED((N, D), jnp.float32),
        ] + [pltpu.SemaphoreType.DMA] * (2 * NB),
    )
    def _sc_scatter(h_hbm, col3_hbm, z_hbm, out_hbm, idx2d, rows_v, acc_sh,
                    *sems):
        c = lax.axis_index("c")
        s = lax.axis_index("s")
        wid = s * NC + c
        base = wid * epw
        lsem = sems[:NB]
        asem = sems[NB:]

        nz = jnp.where(s < NZCH % NS, NZCH // NS + 1, NZCH // NS)

        def zc(k, carry):
            pltpu.async_copy(z_hbm, acc_sh.at[pl.ds((s + k * NS) * CHUNK, CHUNK)],
                             lsem[0])
            return carry

        lax.fori_loop(0, nz, zc, 0)

        def zw(k, carry):
            pltpu.make_async_copy(z_hbm, acc_sh.at[pl.ds(0, CHUNK)],
                                  lsem[0]).wait()
            return carry

        lax.fori_loop(0, nz, zw, 0)
        plsc.subcore_barrier()

        def fire_l(j, b):
            pltpu.async_copy(h_hbm.at[pl.ds(base + j * CHUNK, CHUNK)],
                             rows_v.at[b], lsem[b])

        def wait_l(b):
            pltpu.make_async_copy(h_hbm.at[pl.ds(base, CHUNK)], rows_v.at[b],
                                  lsem[b]).wait()

        def fire_a(j, b):
            pltpu.async_copy(rows_v.at[b], acc_sh.at[idx2d.at[j]], asem[b],
                             add=True)

        def wait_a(b):
            pltpu.make_async_copy(rows_v.at[b], acc_sh.at[idx2d.at[0]],
                                  asem[b]).wait()

        pltpu.sync_copy(col3_hbm.at[wid], idx2d)
        _ring(nchunk, fire_l, wait_l, fire_a, wait_a)
        plsc.subcore_barrier()

        def wb(k, carry):
            r0 = (s + k * NS) * CHUNK
            pltpu.async_copy(acc_sh.at[pl.ds(r0, CHUNK)],
                             out_hbm.at[c, pl.ds(r0, CHUNK)], lsem[0])
            return carry

        lax.fori_loop(0, nz, wb, 0)

        def wbw(k, carry):
            pltpu.make_async_copy(acc_sh.at[pl.ds(0, CHUNK)],
                                  out_hbm.at[c, pl.ds(0, CHUNK)], lsem[0]).wait()
            return carry

        lax.fori_loop(0, nz, wbw, 0)

    return _sc_scatter


def _final_body(x_ref, pa_ref, pb_ref, w1_ref, b1_ref, g2_ref, bb2_ref,
                w2_ref, b2_ref, eps_ref, bg_ref, bb_ref, o_ref):
    x = x_ref[...]
    agg = (pa_ref[0] + pa_ref[1]) + (pb_ref[0] + pb_ref[1])
    out = (1.0 + eps_ref[0, 0]) * x + agg
    t = jnp.dot(out, w1_ref[...], preferred_element_type=jnp.float32) + b1_ref[...]
    mu = jnp.mean(t, axis=1, keepdims=True)
    var = jnp.mean((t - mu) ** 2, axis=1, keepdims=True)
    t = jnp.maximum((t - mu) / jnp.sqrt(var + 1e-5) * g2_ref[...] + bb2_ref[...], 0.0)
    y = jnp.dot(t, w2_ref[...], preferred_element_type=jnp.float32) + b2_ref[...] + 2.0 * x
    m = jnp.mean(y, axis=0, keepdims=True)
    v = jnp.mean((y - m) ** 2, axis=0, keepdims=True)
    o_ref[...] = (y - m) / jnp.sqrt(v + 1e-5) * bg_ref[...] + bb_ref[...]


def _final(x, parts_a, parts_b, w1, b1, g2, bb2, w2, b2, eps, bg, bb):
    return pl.pallas_call(
        _final_body,
        out_shape=jax.ShapeDtypeStruct((N, D), jnp.float32),
    )(x, parts_a, parts_b, w1, b1, g2, bb2, w2, b2, eps, bg, bb)


def kernel(x, edge_index, edge_attr, W_en, b_en, ln1_g, ln1_b, W_m1, b_m1,
           ln2_g, ln2_b, W_m2, b_m2, eps, bn_g, bn_b):
    row = edge_index[0]
    col = edge_index[1]
    row_a = row[:E_A].reshape(NW, NCH_A, CHUNK)
    row_b = row[E_A:].reshape(NW, NCH_B, CHUNK)
    col_a = col[:E_A].reshape(NW, NCH_A, CHUNK)
    col_b = col[E_A:].reshape(NW, NCH_B, CHUNK)
    w_top = W_en[:D]
    w_bot = W_en[D:]
    g1 = ln1_g.reshape(1, D)
    b1 = ln1_b.reshape(1, D)

    zeros = jnp.zeros((CHUNK, D), jnp.float32)
    p = _node_proj(x, w_top, b_en.reshape(1, D))
    g_a = _make_sc_gather(NCH_A)(p, row_a)
    g_b = _make_sc_gather(NCH_B)(p, row_b)
    h_a = _edge_mlp(g_a, edge_attr, w_bot, g1, b1, E_A // BR, 0)
    h_b = _edge_mlp(g_b, edge_attr, w_bot, g1, b1, E_B // BR, E_A // BR)
    parts_a = _make_sc_scatter(NCH_A)(h_a, col_a, zeros)
    parts_b = _make_sc_scatter(NCH_B)(h_b, col_b, zeros)
    return _final(x, parts_a, parts_b, W_m1, b_m1.reshape(1, D),
                  ln2_g.reshape(1, D), ln2_b.reshape(1, D), W_m2,
                  b_m2.reshape(1, D), eps.reshape(1, 1), bn_g.reshape(1, D),
                  bn_b.reshape(1, D))


# speedup vs baseline: 1.0496x; 1.0496x over previous
import functools

import jax
import jax.numpy as jnp
from jax import lax
from jax.experimental import pallas as pl
from jax.experimental.pallas import tpu as pltpu
from jax.experimental.pallas import tpu_sc as plsc

N = 10000
E = 320000
D = 128

NC = 2
NS = 16
NW = NC * NS
CHUNK = 80
NZCH = N // CHUNK

NCH_A = 63
NCH_B = 62
E_A = NW * NCH_A * CHUNK
E_B = E - E_A

BR = 2560


def _proj_body(x_ref, w_ref, b_ref, o_ref):
    o_ref[...] = jnp.dot(x_ref[...], w_ref[...],
                         preferred_element_type=jnp.float32) + b_ref[...]


def _node_proj(x, w_top, b_en):
    return pl.pallas_call(
        _proj_body,
        out_shape=jax.ShapeDtypeStruct((N, D), jnp.float32),
    )(x, w_top, b_en)


NB = 4


def _ring(nchunk, fire_in, wait_in, fire_out, wait_out):
    for b in range(NB - 1):
        fire_in(b, b)
    full = nchunk // NB
    rem = nchunk % NB

    def step(j, u, i):
        wait_in(u)
        fire_out(j, u)
        pb = (u - 1) % NB
        if u == 0:
            if i is None:
                wait_out(pb)
            else:
                @pl.when(i > 0)
                def _():
                    wait_out(pb)
        else:
            wait_out(pb)
        if i is None:
            if j + NB - 1 < nchunk:
                fire_in(j + NB - 1, pb)
        elif u < rem:
            fire_in(j + NB - 1, pb)
        else:
            @pl.when(j + NB - 1 < nchunk)
            def _():
                fire_in(j + NB - 1, pb)

    def body(i, carry):
        for u in range(NB):
            step(i * NB + u, u, i)
        return carry

    lax.fori_loop(0, full, body, 0)
    for u in range(rem):
        step(full * NB + u, u, None)
    wait_out((nchunk - 1) % NB)


@functools.cache
def _make_sc_gather(nchunk):
    mesh = plsc.VectorSubcoreMesh(core_axis_name="c", subcore_axis_name="s")
    epw = nchunk * CHUNK

    @functools.partial(
        pl.kernel,
        mesh=mesh,
        out_type=jax.ShapeDtypeStruct((NW * epw, D), jnp.float32),
        scratch_types=[
            pltpu.VMEM((nchunk, CHUNK), jnp.int32),
            pltpu.VMEM((NB, CHUNK, D), jnp.float32),
            pltpu.VMEM_SHARED((N, D), jnp.float32),
        ] + [pltpu.SemaphoreType.DMA] * (2 * NB),
    )
    def _sc_gather(p_hbm, row3_hbm, out_hbm, idx2d, rows_v, tab_sh, *sems):
        s = lax.axis_index("s")
        wid = s * NC + lax.axis_index("c")
        base = wid * epw
        gsem = sems[:NB]
        ssem = sems[NB:]

        nz = jnp.where(s < NZCH % NS, NZCH // NS + 1, NZCH // NS)

        def st(k, carry):
            r0 = (s + k * NS) * CHUNK
            pltpu.sync_copy(p_hbm.at[pl.ds(r0, CHUNK)], rows_v.at[0])
            pltpu.sync_copy(rows_v.at[0], tab_sh.at[pl.ds(r0, CHUNK)])
            return carry

        lax.fori_loop(0, nz, st, 0)
        plsc.subcore_barrier()

        def fire_g(j, b):
            pltpu.async_copy(tab_sh.at[idx2d.at[j]], rows_v.at[b], gsem[b])

        def wait_g(b):
            pltpu.make_async_copy(tab_sh.at[idx2d.at[0]], rows_v.at[b],
                                  gsem[b]).wait()

        def fire_s(j, b):
            pltpu.async_copy(rows_v.at[b],
                             out_hbm.at[pl.ds(base + j * CHUNK, CHUNK)], ssem[b])

        def wait_s(b):
            pltpu.make_async_copy(rows_v.at[b], out_hbm.at[pl.ds(base, CHUNK)],
                                  ssem[b]).wait()

        pltpu.sync_copy(row3_hbm.at[wid], idx2d)
        _ring(nchunk, fire_g, wait_g, fire_s, wait_s)

    return _sc_gather


def _edge_body(g_ref, a_ref, w_ref, g1_ref, b1_ref, o_ref):
    t = g_ref[...] + jnp.dot(a_ref[...], w_ref[...],
                             preferred_element_type=jnp.float32)
    jm = jnp.full((D, D), 1.0 / D, dtype=jnp.float32)
    mu = jnp.dot(t, jm, preferred_element_type=jnp.float32)
    m2 = jnp.dot(t * t, jm, preferred_element_type=jnp.float32)
    var = m2 - mu * mu
    t = (t - mu) / jnp.sqrt(var + 1e-5) * g1_ref[...] + b1_ref[...]
    o_ref[...] = jnp.maximum(t, 0.0)


def _edge_mlp(g, a, w_bot, g1, b1, steps, off):
    return pl.pallas_call(
        _edge_body,
        grid=(steps,),
        in_specs=[
            pl.BlockSpec((BR, D), lambda i: (i, 0)),
            pl.BlockSpec((BR, D), lambda i: (i + off, 0)),
            pl.BlockSpec((D, D), lambda i: (0, 0)),
            pl.BlockSpec((1, D), lambda i: (0, 0)),
            pl.BlockSpec((1, D), lambda i: (0, 0)),
        ],
        out_specs=pl.BlockSpec((BR, D), lambda i: (i, 0)),
        out_shape=jax.ShapeDtypeStruct((steps * BR, D), jnp.float32),
    )(g, a, w_bot, g1, b1)


@functools.cache
def _make_sc_scatter(nchunk):
    mesh = plsc.VectorSubcoreMesh(core_axis_name="c", subcore_axis_name="s")
    epw = nchunk * CHUNK

    @functools.partial(
        pl.kernel,
        mesh=mesh,
        out_type=jax.ShapeDtypeStruct((NC, N, D), jnp.float32),
        scratch_types=[
            pltpu.VMEM((nchunk, CHUNK), jnp.int32),
            pltpu.VMEM((NB, CHUNK, D), jnp.float32),
            pltpu.VMEM_SHARED((N, D), jnp.float32),
        ] + [pltpu.SemaphoreType.DMA] * (2 * NB),
    )
    def _sc_scatter(h_hbm, col3_hbm, out_hbm, idx2d, rows_v, acc_sh, *sems):
        c = lax.axis_index("c")
        s = lax.axis_index("s")
        wid = s * NC + c
        base = wid * epw
        lsem = sems[:NB]
        asem = sems[NB:]

        zv = jnp.zeros((16,), jnp.float32)

        def zb(i, carry):
            r = i // (D // 16)
            q = (i % (D // 16)) * 16
            rows_v[0, r, pl.ds(q, 16)] = zv
            return carry

        lax.fori_loop(0, CHUNK * (D // 16), zb, 0)

        nz = jnp.where(s < NZCH % NS, NZCH // NS + 1, NZCH // NS)

        def zc(k, carry):
            pltpu.sync_copy(rows_v.at[0],
                            acc_sh.at[pl.ds((s + k * NS) * CHUNK, CHUNK)])
            return carry

        lax.fori_loop(0, nz, zc, 0)
        plsc.subcore_barrier()

        def fire_l(j, b):
            pltpu.async_copy(h_hbm.at[pl.ds(base + j * CHUNK, CHUNK)],
                             rows_v.at[b], lsem[b])

        def wait_l(b):
            pltpu.make_async_copy(h_hbm.at[pl.ds(base, CHUNK)], rows_v.at[b],
                                  lsem[b]).wait()

        def fire_a(j, b):
            pltpu.async_copy(rows_v.at[b], acc_sh.at[idx2d.at[j]], asem[b],
                             add=True)

        def wait_a(b):
            pltpu.make_async_copy(rows_v.at[b], acc_sh.at[idx2d.at[0]],
                                  asem[b]).wait()

        pltpu.sync_copy(col3_hbm.at[wid], idx2d)
        _ring(nchunk, fire_l, wait_l, fire_a, wait_a)
        plsc.subcore_barrier()

        def wb(k, carry):
            r0 = (s + k * NS) * CHUNK
            pltpu.sync_copy(acc_sh.at[pl.ds(r0, CHUNK)], rows_v.at[0])
            pltpu.sync_copy(rows_v.at[0], out_hbm.at[c, pl.ds(r0, CHUNK)])
            return carry

        lax.fori_loop(0, nz, wb, 0)

    return _sc_scatter


def _final_body(x_ref, pa_ref, pb_ref, w1_ref, b1_ref, g2_ref, bb2_ref,
                w2_ref, b2_ref, eps_ref, bg_ref, bb_ref, o_ref):
    x = x_ref[...]
    agg = (pa_ref[0] + pa_ref[1]) + (pb_ref[0] + pb_ref[1])
    out = (1.0 + eps_ref[0, 0]) * x + agg
    t = jnp.dot(out, w1_ref[...], preferred_element_type=jnp.float32) + b1_ref[...]
    mu = jnp.mean(t, axis=1, keepdims=True)
    var = jnp.mean((t - mu) ** 2, axis=1, keepdims=True)
    t = jnp.maximum((t - mu) / jnp.sqrt(var + 1e-5) * g2_ref[...] + bb2_ref[...], 0.0)
    y = jnp.dot(t, w2_ref[...], preferred_element_type=jnp.float32) + b2_ref[...] + 2.0 * x
    m = jnp.mean(y, axis=0, keepdims=True)
    v = jnp.mean((y - m) ** 2, axis=0, keepdims=True)
    o_ref[...] = (y - m) / jnp.sqrt(v + 1e-5) * bg_ref[...] + bb_ref[...]


def _final(x, parts_a, parts_b, w1, b1, g2, bb2, w2, b2, eps, bg, bb):
    return pl.pallas_call(
        _final_body,
        out_shape=jax.ShapeDtypeStruct((N, D), jnp.float32),
    )(x, parts_a, parts_b, w1, b1, g2, bb2, w2, b2, eps, bg, bb)


def kernel(x, edge_index, edge_attr, W_en, b_en, ln1_g, ln1_b, W_m1, b_m1,
           ln2_g, ln2_b, W_m2, b_m2, eps, bn_g, bn_b):
    row = edge_index[0]
    col = edge_index[1]
    row_a = row[:E_A].reshape(NW, NCH_A, CHUNK)
    row_b = row[E_A:].reshape(NW, NCH_B, CHUNK)
    col_a = col[:E_A].reshape(NW, NCH_A, CHUNK)
    col_b = col[E_A:].reshape(NW, NCH_B, CHUNK)
    w_top = W_en[:D]
    w_bot = W_en[D:]
    g1 = ln1_g.reshape(1, D)
    b1 = ln1_b.reshape(1, D)

    p = _node_proj(x, w_top, b_en.reshape(1, D))
    g_a = _make_sc_gather(NCH_A)(p, row_a)
    g_b = _make_sc_gather(NCH_B)(p, row_b)
    h_a = _edge_mlp(g_a, edge_attr, w_bot, g1, b1, E_A // BR, 0)
    h_b = _edge_mlp(g_b, edge_attr, w_bot, g1, b1, E_B // BR, E_A // BR)
    parts_a = _make_sc_scatter(NCH_A)(h_a, col_a)
    parts_b = _make_sc_scatter(NCH_B)(h_b, col_b)
    return _final(x, parts_a, parts_b, W_m1, b_m1.reshape(1, D),
                  ln2_g.reshape(1, D), ln2_b.reshape(1, D), W_m2,
                  b_m2.reshape(1, D), eps.reshape(1, 1), bn_g.reshape(1, D),
                  bn_b.reshape(1, D))


# speedup vs baseline: 1.0737x; 1.0230x over previous
import functools

import jax
import jax.numpy as jnp
from jax import lax
from jax.experimental import pallas as pl
from jax.experimental.pallas import tpu as pltpu
from jax.experimental.pallas import tpu_sc as plsc

N = 10000
E = 320000
D = 128

NC = 2
NS = 16
NW = NC * NS
CHUNK = 80
NZCH = N // CHUNK

NCHS = (42, 42, 41)
E_PARTS = tuple(NW * n * CHUNK for n in NCHS)

BR = 2560


def _proj_body(x_ref, w_ref, b_ref, o_ref):
    o_ref[...] = jnp.dot(x_ref[...], w_ref[...],
                         preferred_element_type=jnp.float32) + b_ref[...]


def _node_proj(x, w_top, b_en):
    return pl.pallas_call(
        _proj_body,
        out_shape=jax.ShapeDtypeStruct((N, D), jnp.float32),
    )(x, w_top, b_en)


NB = 4


def _ring(nchunk, fire_in, wait_in, fire_out, wait_out):
    for b in range(NB - 1):
        fire_in(b, b)
    full = nchunk // NB
    rem = nchunk % NB

    def step(j, u, i):
        wait_in(u)
        fire_out(j, u)
        pb = (u - 1) % NB
        if u == 0:
            if i is None:
                wait_out(pb)
            else:
                @pl.when(i > 0)
                def _():
                    wait_out(pb)
        else:
            wait_out(pb)
        if i is None:
            if j + NB - 1 < nchunk:
                fire_in(j + NB - 1, pb)
        elif u < rem:
            fire_in(j + NB - 1, pb)
        else:
            @pl.when(j + NB - 1 < nchunk)
            def _():
                fire_in(j + NB - 1, pb)

    def body(i, carry):
        for u in range(NB):
            step(i * NB + u, u, i)
        return carry

    lax.fori_loop(0, full, body, 0)
    for u in range(rem):
        step(full * NB + u, u, None)
    wait_out((nchunk - 1) % NB)


@functools.cache
def _make_sc_gather(nchunk):
    mesh = plsc.VectorSubcoreMesh(core_axis_name="c", subcore_axis_name="s")
    epw = nchunk * CHUNK

    @functools.partial(
        pl.kernel,
        mesh=mesh,
        out_type=jax.ShapeDtypeStruct((NW * epw, D), jnp.float32),
        scratch_types=[
            pltpu.VMEM((nchunk, CHUNK), jnp.int32),
            pltpu.VMEM((NB, CHUNK, D), jnp.float32),
            pltpu.VMEM_SHARED((N, D), jnp.float32),
        ] + [pltpu.SemaphoreType.DMA] * (2 * NB),
    )
    def _sc_gather(p_hbm, row3_hbm, out_hbm, idx2d, rows_v, tab_sh, *sems):
        s = lax.axis_index("s")
        wid = s * NC + lax.axis_index("c")
        base = wid * epw
        gsem = sems[:NB]
        ssem = sems[NB:]

        nz = jnp.where(s < NZCH % NS, NZCH // NS + 1, NZCH // NS)

        def st(k, carry):
            r0 = (s + k * NS) * CHUNK
            pltpu.sync_copy(p_hbm.at[pl.ds(r0, CHUNK)], rows_v.at[0])
            pltpu.sync_copy(rows_v.at[0], tab_sh.at[pl.ds(r0, CHUNK)])
            return carry

        lax.fori_loop(0, nz, st, 0)
        plsc.subcore_barrier()

        def fire_g(j, b):
            pltpu.async_copy(tab_sh.at[idx2d.at[j]], rows_v.at[b], gsem[b])

        def wait_g(b):
            pltpu.make_async_copy(tab_sh.at[idx2d.at[0]], rows_v.at[b],
                                  gsem[b]).wait()

        def fire_s(j, b):
            pltpu.async_copy(rows_v.at[b],
                             out_hbm.at[pl.ds(base + j * CHUNK, CHUNK)], ssem[b])

        def wait_s(b):
            pltpu.make_async_copy(rows_v.at[b], out_hbm.at[pl.ds(base, CHUNK)],
                                  ssem[b]).wait()

        pltpu.sync_copy(row3_hbm.at[wid], idx2d)
        _ring(nchunk, fire_g, wait_g, fire_s, wait_s)

    return _sc_gather


def _edge_body(g_ref, a_ref, w_ref, g1_ref, b1_ref, o_ref):
    t = g_ref[...] + jnp.dot(a_ref[...], w_ref[...],
                             preferred_element_type=jnp.float32)
    jm = jnp.full((D, D), 1.0 / D, dtype=jnp.float32)
    mu = jnp.dot(t, jm, preferred_element_type=jnp.float32)
    m2 = jnp.dot(t * t, jm, preferred_element_type=jnp.float32)
    var = m2 - mu * mu
    t = (t - mu) / jnp.sqrt(var + 1e-5) * g1_ref[...] + b1_ref[...]
    o_ref[...] = jnp.maximum(t, 0.0)


def _edge_mlp(g, a, w_bot, g1, b1, steps, off):
    return pl.pallas_call(
        _edge_body,
        grid=(steps,),
        in_specs=[
            pl.BlockSpec((BR, D), lambda i: (i, 0)),
            pl.BlockSpec((BR, D), lambda i: (i + off, 0)),
            pl.BlockSpec((D, D), lambda i: (0, 0)),
            pl.BlockSpec((1, D), lambda i: (0, 0)),
            pl.BlockSpec((1, D), lambda i: (0, 0)),
        ],
        out_specs=pl.BlockSpec((BR, D), lambda i: (i, 0)),
        out_shape=jax.ShapeDtypeStruct((steps * BR, D), jnp.float32),
    )(g, a, w_bot, g1, b1)


@functools.cache
def _make_sc_scatter(nchunk):
    mesh = plsc.VectorSubcoreMesh(core_axis_name="c", subcore_axis_name="s")
    epw = nchunk * CHUNK

    @functools.partial(
        pl.kernel,
        mesh=mesh,
        out_type=jax.ShapeDtypeStruct((NC, N, D), jnp.float32),
        scratch_types=[
            pltpu.VMEM((nchunk, CHUNK), jnp.int32),
            pltpu.VMEM((NB, CHUNK, D), jnp.float32),
            pltpu.VMEM_SHARED((N, D), jnp.float32),
        ] + [pltpu.SemaphoreType.DMA] * (2 * NB),
    )
    def _sc_scatter(h_hbm, col3_hbm, out_hbm, idx2d, rows_v, acc_sh, *sems):
        c = lax.axis_index("c")
        s = lax.axis_index("s")
        wid = s * NC + c
        base = wid * epw
        lsem = sems[:NB]
        asem = sems[NB:]

        zv = jnp.zeros((16,), jnp.float32)

        def zb(i, carry):
            r = i // (D // 16)
            q = (i % (D // 16)) * 16
            rows_v[0, r, pl.ds(q, 16)] = zv
            return carry

        lax.fori_loop(0, CHUNK * (D // 16), zb, 0)

        nz = jnp.where(s < NZCH % NS, NZCH // NS + 1, NZCH // NS)

        def zc(k, carry):
            pltpu.sync_copy(rows_v.at[0],
                            acc_sh.at[pl.ds((s + k * NS) * CHUNK, CHUNK)])
            return carry

        lax.fori_loop(0, nz, zc, 0)
        plsc.subcore_barrier()

        def fire_l(j, b):
            pltpu.async_copy(h_hbm.at[pl.ds(base + j * CHUNK, CHUNK)],
                             rows_v.at[b], lsem[b])

        def wait_l(b):
            pltpu.make_async_copy(h_hbm.at[pl.ds(base, CHUNK)], rows_v.at[b],
                                  lsem[b]).wait()

        def fire_a(j, b):
            pltpu.async_copy(rows_v.at[b], acc_sh.at[idx2d.at[j]], asem[b],
                             add=True)

        def wait_a(b):
            pltpu.make_async_copy(rows_v.at[b], acc_sh.at[idx2d.at[0]],
                                  asem[b]).wait()

        pltpu.sync_copy(col3_hbm.at[wid], idx2d)
        _ring(nchunk, fire_l, wait_l, fire_a, wait_a)
        plsc.subcore_barrier()

        def wb(k, carry):
            r0 = (s + k * NS) * CHUNK
            pltpu.sync_copy(acc_sh.at[pl.ds(r0, CHUNK)], rows_v.at[0])
            pltpu.sync_copy(rows_v.at[0], out_hbm.at[c, pl.ds(r0, CHUNK)])
            return carry

        lax.fori_loop(0, nz, wb, 0)

    return _sc_scatter


def _final_body(x_ref, pa_ref, pb_ref, pc_ref, w1_ref, b1_ref, g2_ref,
                bb2_ref, w2_ref, b2_ref, eps_ref, bg_ref, bb_ref, o_ref):
    x = x_ref[...]
    agg = ((pa_ref[0] + pa_ref[1]) + (pb_ref[0] + pb_ref[1])
           + (pc_ref[0] + pc_ref[1]))
    out = (1.0 + eps_ref[0, 0]) * x + agg
    t = jnp.dot(out, w1_ref[...], preferred_element_type=jnp.float32) + b1_ref[...]
    mu = jnp.mean(t, axis=1, keepdims=True)
    var = jnp.mean((t - mu) ** 2, axis=1, keepdims=True)
    t = jnp.maximum((t - mu) / jnp.sqrt(var + 1e-5) * g2_ref[...] + bb2_ref[...], 0.0)
    y = jnp.dot(t, w2_ref[...], preferred_element_type=jnp.float32) + b2_ref[...] + 2.0 * x
    m = jnp.mean(y, axis=0, keepdims=True)
    v = jnp.mean((y - m) ** 2, axis=0, keepdims=True)
    o_ref[...] = (y - m) / jnp.sqrt(v + 1e-5) * bg_ref[...] + bb_ref[...]


def _final(x, parts, w1, b1, g2, bb2, w2, b2, eps, bg, bb):
    return pl.pallas_call(
        _final_body,
        out_shape=jax.ShapeDtypeStruct((N, D), jnp.float32),
    )(x, *parts, w1, b1, g2, bb2, w2, b2, eps, bg, bb)


def kernel(x, edge_index, edge_attr, W_en, b_en, ln1_g, ln1_b, W_m1, b_m1,
           ln2_g, ln2_b, W_m2, b_m2, eps, bn_g, bn_b):
    row = edge_index[0]
    col = edge_index[1]
    w_top = W_en[:D]
    w_bot = W_en[D:]
    g1 = ln1_g.reshape(1, D)
    b1 = ln1_b.reshape(1, D)

    p = _node_proj(x, w_top, b_en.reshape(1, D))

    offs = [0]
    for ep in E_PARTS:
        offs.append(offs[-1] + ep)
    gs = []
    for k, nch in enumerate(NCHS):
        row_k = row[offs[k]:offs[k + 1]].reshape(NW, nch, CHUNK)
        gs.append(_make_sc_gather(nch)(p, row_k))
    hs = [_edge_mlp(gs[k], edge_attr, w_bot, g1, b1, E_PARTS[k] // BR,
                    offs[k] // BR) for k in range(len(NCHS))]
    parts = []
    for k, nch in enumerate(NCHS):
        col_k = col[offs[k]:offs[k + 1]].reshape(NW, nch, CHUNK)
        parts.append(_make_sc_scatter(nch)(hs[k], col_k))
    return _final(x, parts, W_m1, b_m1.reshape(1, D),
                  ln2_g.reshape(1, D), ln2_b.reshape(1, D), W_m2,
                  b_m2.reshape(1, D), eps.reshape(1, 1), bn_g.reshape(1, D),
                  bn_b.reshape(1, D))
